# baseline (speedup 1.0000x reference)
import functools

import jax
import jax.numpy as jnp
from jax import lax
from jax.experimental import pallas as pl
from jax.experimental.pallas import tpu as pltpu
from jax.experimental.pallas import tpu_sc as plsc

N = 10000
E = 320000
D = 128
G = 256

NC = 2
NS = 16
EW = E // (NC * NS)
CHUNK = 80
NCH = EW // CHUNK
NPAD = 10240
RS = NPAD // NS

MBLK = 1000
NBLK = N // MBLK


def _make_sc_scatter(width):
    mesh = plsc.VectorSubcoreMesh(core_axis_name="c", subcore_axis_name="s")

    def _gather(g_hbm, sidx, rows, sem):
        return pltpu.async_copy(g_hbm.at[sidx], rows, sem)

    @functools.partial(
        pl.kernel,
        out_type=jax.ShapeDtypeStruct((NC, NPAD, width), jnp.float32),
        mesh=mesh,
        scratch_types=[
            pltpu.VMEM((CHUNK,), jnp.int32),
            pltpu.VMEM((CHUNK,), jnp.int32),
            pltpu.VMEM((NCH, CHUNK), jnp.int32),
            pltpu.VMEM((CHUNK, width), jnp.float32),
            pltpu.VMEM((CHUNK, width), jnp.float32),
            pltpu.VMEM_SHARED((NPAD, width), jnp.float32),
            pltpu.SemaphoreType.DMA,
            pltpu.SemaphoreType.DMA,
            pltpu.SemaphoreType.DMA,
            pltpu.SemaphoreType.DMA,
            pltpu.SemaphoreType.DMA,
            pltpu.SemaphoreType.DMA,
        ],
    )
    def sc_scatter(g_hbm, src_hbm, dst_hbm, zero_hbm, out_hbm,
                   sidx0, sidx1, didx, rows0, rows1, acc,
                   sem0, sem1, isem0, isem1, ssem0, ssem1):
        cid = lax.axis_index("c")
        sid = lax.axis_index("s")
        wid = cid * NS + sid
        pltpu.sync_copy(zero_hbm, acc.at[pl.ds(sid * RS, RS)])
        pltpu.sync_copy(dst_hbm.at[wid], didx)
        plsc.subcore_barrier()

        pltpu.sync_copy(src_hbm.at[wid, 0], sidx0)
        pltpu.sync_copy(src_hbm.at[wid, 1], sidx1)
        _gather(g_hbm, sidx0, rows0, sem0)
        _gather(g_hbm, sidx1, rows1, sem1)

        def body(j, carry):
            i0 = 2 * j
            pltpu.make_async_copy(g_hbm.at[sidx0], rows0, sem0).wait()
            pltpu.async_copy(rows0, acc.at[didx.at[i0]], ssem0, add=True)

            @pl.when(i0 + 2 < NCH)
            def _():
                pltpu.async_copy(src_hbm.at[wid, i0 + 2], sidx0, isem0)

            pltpu.make_async_copy(g_hbm.at[sidx1], rows1, sem1).wait()
            pltpu.async_copy(rows1, acc.at[didx.at[i0 + 1]], ssem1, add=True)

            @pl.when(i0 + 3 < NCH)
            def _():
                pltpu.async_copy(src_hbm.at[wid, i0 + 3], sidx1, isem1)

            pltpu.make_async_copy(rows0, acc.at[didx.at[i0]], ssem0).wait()

            @pl.when(i0 + 2 < NCH)
            def _():
                pltpu.make_async_copy(src_hbm.at[wid, 0], sidx0, isem0).wait()
                _gather(g_hbm, sidx0, rows0, sem0)

            pltpu.make_async_copy(rows1, acc.at[didx.at[i0 + 1]],
                                  ssem1).wait()

            @pl.when(i0 + 3 < NCH)
            def _():
                pltpu.make_async_copy(src_hbm.at[wid, 0], sidx1, isem1).wait()
                _gather(g_hbm, sidx1, rows1, sem1)

            return carry

        lax.fori_loop(0, (NCH - 1) // 2, body, 0)
        pltpu.make_async_copy(g_hbm.at[sidx0], rows0, sem0).wait()
        pltpu.sync_copy(rows0, acc.at[didx.at[NCH - 1]], add=True)

        plsc.subcore_barrier()
        pltpu.sync_copy(acc.at[pl.ds(sid * RS, RS)],
                        out_hbm.at[cid, pl.ds(sid * RS, RS)])

    return sc_scatter


_sc_scatter_rows = _make_sc_scatter(D)


_HROW = NPAD // D


def _make_sc_deg():
    mesh = plsc.VectorSubcoreMesh(core_axis_name="c", subcore_axis_name="s")

    @functools.partial(
        pl.kernel,
        out_type=jax.ShapeDtypeStruct((NC * NS, _HROW, D), jnp.float32),
        mesh=mesh,
        compiler_params=pltpu.CompilerParams(needs_layout_passes=False),
        scratch_types=[
            pltpu.VMEM((EW,), jnp.int32),
            pltpu.VMEM((_HROW, D), jnp.float32),
        ],
    )
    def sc_deg(dst_hbm, zero_hbm, out_hbm, didx, hist):
        cid = lax.axis_index("c")
        sid = lax.axis_index("s")
        wid = cid * NS + sid
        pltpu.sync_copy(zero_hbm.at[pl.ds(0, _HROW)], hist)
        pltpu.sync_copy(dst_hbm.at[pl.ds(wid * EW, EW)], didx)

        def body(k, carry):
            d = didx[pl.ds(16 * k, 16)]
            cnt, last = plsc.scan_count(d)
            row = lax.shift_right_logical(d, 7)
            col = lax.bitwise_and(d, 127)
            plsc.addupdate_scatter(hist, [row, col],
                                   cnt.astype(jnp.float32), mask=last)
            return carry

        lax.fori_loop(0, EW // 16, body, 0)
        pltpu.sync_copy(hist, out_hbm.at[wid])

    return sc_deg


_sc_deg = _make_sc_deg()


def _tc_dinv(degp):
    def body(degp_ref, out_ref):
        acc = degp_ref[0]
        for t in range(1, NC * NS):
            acc = acc + degp_ref[t]
        out_ref[...] = lax.rsqrt(acc + 1.0)

    return pl.pallas_call(
        body,
        out_shape=jax.ShapeDtypeStruct((_HROW, D), jnp.float32),
    )(degp)


def _tc_first(x, fcW, fcb, W0, dinv):
    def body(x_ref, fcW_ref, fcb_ref, W0_ref, dinv_ref, g_ref):
        h = jnp.dot(x_ref[...], fcW_ref[...],
                    preferred_element_type=jnp.float32) + fcb_ref[...]
        p = jnp.dot(h, W0_ref[...], preferred_element_type=jnp.float32)
        g_ref[...] = dinv_ref[...] * p

    return pl.pallas_call(
        body,
        grid=(NBLK,),
        in_specs=[
            pl.BlockSpec((MBLK, D), lambda i: (i, 0)),
            pl.BlockSpec((D, D), lambda i: (0, 0)),
            pl.BlockSpec((1, D), lambda i: (0, 0)),
            pl.BlockSpec((D, D), lambda i: (0, 0)),
            pl.BlockSpec((MBLK, 1), lambda i: (i, 0)),
        ],
        out_specs=pl.BlockSpec((MBLK, D), lambda i: (i, 0)),
        out_shape=jax.ShapeDtypeStruct((N, D), jnp.float32),
    )(x, fcW, fcb, W0, dinv)


def _tc_layer(Sp, g, dinv, b, W):
    def body(sp_ref, g_ref, dinv_ref, b_ref, W_ref, out_ref):
        h = dinv_ref[...] * (sp_ref[0] + sp_ref[1] + g_ref[...]) + b_ref[...]
        out_ref[...] = dinv_ref[...] * jnp.dot(
            h, W_ref[...], preferred_element_type=jnp.float32)

    return pl.pallas_call(
        body,
        grid=(NBLK,),
        in_specs=[
            pl.BlockSpec((NC, MBLK, D), lambda i: (0, i, 0)),
            pl.BlockSpec((MBLK, D), lambda i: (i, 0)),
            pl.BlockSpec((MBLK, 1), lambda i: (i, 0)),
            pl.BlockSpec((1, D), lambda i: (0, 0)),
            pl.BlockSpec((D, D), lambda i: (0, 0)),
        ],
        out_specs=pl.BlockSpec((MBLK, D), lambda i: (i, 0)),
        out_shape=jax.ShapeDtypeStruct((N, D), jnp.float32),
    )(Sp, g, dinv, b, W)


def _tc_final(Sp, g, dinv, b, batch3):
    def body(sp_ref, g_ref, dinv_ref, b_ref, batch_ref, out_ref):
        i = pl.program_id(0)
        h = dinv_ref[...] * (sp_ref[0] + sp_ref[1] + g_ref[...]) + b_ref[...]
        brow = batch_ref[0]
        oh = (brow == lax.broadcasted_iota(jnp.int32, (G, MBLK), 0)
              ).astype(jnp.float32)
        contrib = jnp.dot(oh, h, preferred_element_type=jnp.float32)

        @pl.when(i == 0)
        def _():
            out_ref[...] = jnp.zeros_like(out_ref)

        out_ref[...] += contrib

    return pl.pallas_call(
        body,
        grid=(NBLK,),
        in_specs=[
            pl.BlockSpec((NC, MBLK, D), lambda i: (0, i, 0)),
            pl.BlockSpec((MBLK, D), lambda i: (i, 0)),
            pl.BlockSpec((MBLK, 1), lambda i: (i, 0)),
            pl.BlockSpec((1, D), lambda i: (0, 0)),
            pl.BlockSpec((1, 1, MBLK), lambda i: (i, 0, 0)),
        ],
        out_specs=pl.BlockSpec((G, D), lambda i: (0, 0)),
        out_shape=jax.ShapeDtypeStruct((G, D), jnp.float32),
    )(Sp, g, dinv, b, batch3)


def kernel(x, edge_index, batch, fc_W, fc_b, W0, b0, W1, b1, W2, b2):
    src = edge_index[0].reshape(NC * NS, NCH, CHUNK)
    dst = edge_index[1].reshape(NC * NS, NCH, CHUNK)
    zeroD = jnp.zeros((RS, D), jnp.float32)

    degp = _sc_deg(edge_index[1], zeroD)
    dinv = _tc_dinv(degp).reshape(NPAD, 1)[:N]
    g0 = _tc_first(x, fc_W, fc_b.reshape(1, D), W0, dinv)
    Sp = _sc_scatter_rows(g0, src, dst, zeroD)
    g1 = _tc_layer(Sp, g0, dinv, b0.reshape(1, D), W1)
    Sp = _sc_scatter_rows(g1, src, dst, zeroD)
    g2 = _tc_layer(Sp, g1, dinv, b1.reshape(1, D), W2)
    Sp = _sc_scatter_rows(g2, src, dst, zeroD)
    return _tc_final(Sp, g2, dinv, b2.reshape(1, D),
                     batch.reshape(NBLK, 1, MBLK))


# speedup vs baseline: 1.1905x; 1.1905x over previous
import functools

import jax
import jax.numpy as jnp
from jax import lax
from jax.experimental import pallas as pl
from jax.experimental.pallas import tpu as pltpu
from jax.experimental.pallas import tpu_sc as plsc

N = 10000
E = 320000
D = 128
G = 256

NC = 2
NS = 16
EW = E // (NC * NS)
CHUNK = 80
NCH = EW // CHUNK
NPAD = 10240
RS = NPAD // NS

MBLK = 1000
NBLK = N // MBLK


def _make_sc_scatter(width):
    mesh = plsc.VectorSubcoreMesh(core_axis_name="c", subcore_axis_name="s")

    def _gather(g_hbm, sidx, rows, sem):
        return pltpu.async_copy(g_hbm.at[sidx], rows, sem)

    @functools.partial(
        pl.kernel,
        out_type=jax.ShapeDtypeStruct((NC, NPAD, width), jnp.float32),
        mesh=mesh,
        scratch_types=[
            pltpu.VMEM((CHUNK,), jnp.int32),
            pltpu.VMEM((CHUNK,), jnp.int32),
            pltpu.VMEM((NCH, CHUNK), jnp.int32),
            pltpu.VMEM((CHUNK, width), jnp.float32),
            pltpu.VMEM((CHUNK, width), jnp.float32),
            pltpu.VMEM_SHARED((NPAD, width), jnp.float32),
            pltpu.SemaphoreType.DMA,
            pltpu.SemaphoreType.DMA,
            pltpu.SemaphoreType.DMA,
            pltpu.SemaphoreType.DMA,
            pltpu.SemaphoreType.DMA,
            pltpu.SemaphoreType.DMA,
        ],
    )
    def sc_scatter(g_hbm, src_hbm, dst_hbm, zero_hbm, out_hbm,
                   sidx0, sidx1, didx, rows0, rows1, acc,
                   sem0, sem1, isem0, isem1, ssem0, ssem1):
        cid = lax.axis_index("c")
        sid = lax.axis_index("s")
        wid = cid * NS + sid
        pltpu.sync_copy(zero_hbm, acc.at[pl.ds(sid * RS, RS)])
        pltpu.sync_copy(dst_hbm.at[wid], didx)
        plsc.subcore_barrier()

        pltpu.sync_copy(src_hbm.at[wid, 0], sidx0)
        pltpu.sync_copy(src_hbm.at[wid, 1], sidx1)
        _gather(g_hbm, sidx0, rows0, sem0)
        _gather(g_hbm, sidx1, rows1, sem1)

        def body(j, carry):
            i0 = 2 * j
            pltpu.make_async_copy(g_hbm.at[sidx0], rows0, sem0).wait()

            @pl.when(i0 + 2 < NCH)
            def _():
                pltpu.async_copy(src_hbm.at[wid, i0 + 2], sidx0, isem0)

            pltpu.sync_copy(rows0, acc.at[didx.at[i0]], add=True)

            @pl.when(i0 + 2 < NCH)
            def _():
                pltpu.make_async_copy(src_hbm.at[wid, 0], sidx0, isem0).wait()
                _gather(g_hbm, sidx0, rows0, sem0)

            pltpu.make_async_copy(g_hbm.at[sidx1], rows1, sem1).wait()

            @pl.when(i0 + 3 < NCH)
            def _():
                pltpu.async_copy(src_hbm.at[wid, i0 + 3], sidx1, isem1)

            pltpu.sync_copy(rows1, acc.at[didx.at[i0 + 1]], add=True)

            @pl.when(i0 + 3 < NCH)
            def _():
                pltpu.make_async_copy(src_hbm.at[wid, 0], sidx1, isem1).wait()
                _gather(g_hbm, sidx1, rows1, sem1)

            return carry

        lax.fori_loop(0, (NCH - 1) // 2, body, 0)
        pltpu.make_async_copy(g_hbm.at[sidx0], rows0, sem0).wait()
        pltpu.sync_copy(rows0, acc.at[didx.at[NCH - 1]], add=True)

        plsc.subcore_barrier()
        pltpu.sync_copy(acc.at[pl.ds(sid * RS, RS)],
                        out_hbm.at[cid, pl.ds(sid * RS, RS)])

    return sc_scatter


_sc_scatter_rows = _make_sc_scatter(D)


_HROW = NPAD // D


def _make_sc_deg():
    mesh = plsc.VectorSubcoreMesh(core_axis_name="c", subcore_axis_name="s")

    @functools.partial(
        pl.kernel,
        out_type=jax.ShapeDtypeStruct((NC * NS, _HROW, D), jnp.float32),
        mesh=mesh,
        compiler_params=pltpu.CompilerParams(needs_layout_passes=False),
        scratch_types=[
            pltpu.VMEM((EW,), jnp.int32),
            pltpu.VMEM((_HROW, D), jnp.float32),
        ],
    )
    def sc_deg(dst_hbm, zero_hbm, out_hbm, didx, hist):
        cid = lax.axis_index("c")
        sid = lax.axis_index("s")
        wid = cid * NS + sid
        pltpu.sync_copy(zero_hbm.at[pl.ds(0, _HROW)], hist)
        pltpu.sync_copy(dst_hbm.at[pl.ds(wid * EW, EW)], didx)

        def body(k, carry):
            d = didx[pl.ds(16 * k, 16)]
            cnt, last = plsc.scan_count(d)
            row = lax.shift_right_logical(d, 7)
            col = lax.bitwise_and(d, 127)
            plsc.addupdate_scatter(hist, [row, col],
                                   cnt.astype(jnp.float32), mask=last)
            return carry

        lax.fori_loop(0, EW // 16, body, 0)
        pltpu.sync_copy(hist, out_hbm.at[wid])

    return sc_deg


_sc_deg = _make_sc_deg()


def _tc_dinv(degp):
    def body(degp_ref, out_ref):
        acc = degp_ref[0]
        for t in range(1, NC * NS):
            acc = acc + degp_ref[t]
        out_ref[...] = lax.rsqrt(acc + 1.0)

    return pl.pallas_call(
        body,
        out_shape=jax.ShapeDtypeStruct((_HROW, D), jnp.float32),
    )(degp)


def _tc_first(x, fcW, fcb, W0, dinv):
    def body(x_ref, fcW_ref, fcb_ref, W0_ref, dinv_ref, g_ref):
        h = jnp.dot(x_ref[...], fcW_ref[...],
                    preferred_element_type=jnp.float32) + fcb_ref[...]
        p = jnp.dot(h, W0_ref[...], preferred_element_type=jnp.float32)
        g_ref[...] = dinv_ref[...] * p

    return pl.pallas_call(
        body,
        grid=(NBLK,),
        in_specs=[
            pl.BlockSpec((MBLK, D), lambda i: (i, 0)),
            pl.BlockSpec((D, D), lambda i: (0, 0)),
            pl.BlockSpec((1, D), lambda i: (0, 0)),
            pl.BlockSpec((D, D), lambda i: (0, 0)),
            pl.BlockSpec((MBLK, 1), lambda i: (i, 0)),
        ],
        out_specs=pl.BlockSpec((MBLK, D), lambda i: (i, 0)),
        out_shape=jax.ShapeDtypeStruct((N, D), jnp.float32),
    )(x, fcW, fcb, W0, dinv)


def _tc_layer(Sp, g, dinv, b, W):
    def body(sp_ref, g_ref, dinv_ref, b_ref, W_ref, out_ref):
        h = dinv_ref[...] * (sp_ref[0] + sp_ref[1] + g_ref[...]) + b_ref[...]
        out_ref[...] = dinv_ref[...] * jnp.dot(
            h, W_ref[...], preferred_element_type=jnp.float32)

    return pl.pallas_call(
        body,
        grid=(NBLK,),
        in_specs=[
            pl.BlockSpec((NC, MBLK, D), lambda i: (0, i, 0)),
            pl.BlockSpec((MBLK, D), lambda i: (i, 0)),
            pl.BlockSpec((MBLK, 1), lambda i: (i, 0)),
            pl.BlockSpec((1, D), lambda i: (0, 0)),
            pl.BlockSpec((D, D), lambda i: (0, 0)),
        ],
        out_specs=pl.BlockSpec((MBLK, D), lambda i: (i, 0)),
        out_shape=jax.ShapeDtypeStruct((N, D), jnp.float32),
    )(Sp, g, dinv, b, W)


def _tc_final(Sp, g, dinv, b, batch3):
    def body(sp_ref, g_ref, dinv_ref, b_ref, batch_ref, out_ref):
        i = pl.program_id(0)
        h = dinv_ref[...] * (sp_ref[0] + sp_ref[1] + g_ref[...]) + b_ref[...]
        brow = batch_ref[0]
        oh = (brow == lax.broadcasted_iota(jnp.int32, (G, MBLK), 0)
              ).astype(jnp.float32)
        contrib = jnp.dot(oh, h, preferred_element_type=jnp.float32)

        @pl.when(i == 0)
        def _():
            out_ref[...] = jnp.zeros_like(out_ref)

        out_ref[...] += contrib

    return pl.pallas_call(
        body,
        grid=(NBLK,),
        in_specs=[
            pl.BlockSpec((NC, MBLK, D), lambda i: (0, i, 0)),
            pl.BlockSpec((MBLK, D), lambda i: (i, 0)),
            pl.BlockSpec((MBLK, 1), lambda i: (i, 0)),
            pl.BlockSpec((1, D), lambda i: (0, 0)),
            pl.BlockSpec((1, 1, MBLK), lambda i: (i, 0, 0)),
        ],
        out_specs=pl.BlockSpec((G, D), lambda i: (0, 0)),
        out_shape=jax.ShapeDtypeStruct((G, D), jnp.float32),
    )(Sp, g, dinv, b, batch3)


def kernel(x, edge_index, batch, fc_W, fc_b, W0, b0, W1, b1, W2, b2):
    src = edge_index[0].reshape(NC * NS, NCH, CHUNK)
    dst = edge_index[1].reshape(NC * NS, NCH, CHUNK)
    zeroD = jnp.zeros((RS, D), jnp.float32)

    degp = _sc_deg(edge_index[1], zeroD)
    dinv = _tc_dinv(degp).reshape(NPAD, 1)[:N]
    g0 = _tc_first(x, fc_W, fc_b.reshape(1, D), W0, dinv)
    Sp = _sc_scatter_rows(g0, src, dst, zeroD)
    g1 = _tc_layer(Sp, g0, dinv, b0.reshape(1, D), W1)
    Sp = _sc_scatter_rows(g1, src, dst, zeroD)
    g2 = _tc_layer(Sp, g1, dinv, b1.reshape(1, D), W2)
    Sp = _sc_scatter_rows(g2, src, dst, zeroD)
    return _tc_final(Sp, g2, dinv, b2.reshape(1, D),
                     batch.reshape(NBLK, 1, MBLK))


# speedup vs baseline: 1.1919x; 1.0011x over previous
import functools

import jax
import jax.numpy as jnp
from jax import lax
from jax.experimental import pallas as pl
from jax.experimental.pallas import tpu as pltpu
from jax.experimental.pallas import tpu_sc as plsc

N = 10000
E = 320000
D = 128
G = 256

NC = 2
NS = 16
EW = E // (NC * NS)
CHUNK = 80
NCH = EW // CHUNK
NPAD = 10240
RS = NPAD // NS

MBLK = 1000
NBLK = N // MBLK


def _make_sc_scatter(width):
    mesh = plsc.VectorSubcoreMesh(core_axis_name="c", subcore_axis_name="s")

    def _gather(g_hbm, sidx, rows, sem):
        return pltpu.async_copy(g_hbm.at[sidx], rows, sem)

    @functools.partial(
        pl.kernel,
        out_type=jax.ShapeDtypeStruct((NC, NPAD, width), jnp.float32),
        mesh=mesh,
        scratch_types=[
            pltpu.VMEM((CHUNK,), jnp.int32),
            pltpu.VMEM((CHUNK,), jnp.int32),
            pltpu.VMEM((NCH, CHUNK), jnp.int32),
            pltpu.VMEM((CHUNK, width), jnp.float32),
            pltpu.VMEM((CHUNK, width), jnp.float32),
            pltpu.VMEM_SHARED((NPAD, width), jnp.float32),
            pltpu.SemaphoreType.DMA,
            pltpu.SemaphoreType.DMA,
            pltpu.SemaphoreType.DMA,
            pltpu.SemaphoreType.DMA,
        ],
    )
    def sc_scatter(g_hbm, src_hbm, dst_hbm, zero_hbm, out_hbm,
                   sidx0, sidx1, didx, rows0, rows1, acc,
                   sem0, sem1, isem0, isem1):
        cid = lax.axis_index("c")
        sid = lax.axis_index("s")
        wid = cid * NS + sid
        pltpu.sync_copy(zero_hbm, acc.at[pl.ds(sid * RS, RS)])
        pltpu.sync_copy(dst_hbm.at[wid], didx)
        plsc.subcore_barrier()

        pltpu.sync_copy(src_hbm.at[wid, 0], sidx0)
        pltpu.sync_copy(src_hbm.at[wid, 1], sidx1)
        _gather(g_hbm, sidx0, rows0, sem0)
        _gather(g_hbm, sidx1, rows1, sem1)

        def body(j, carry):
            i0 = 2 * j
            pltpu.make_async_copy(g_hbm.at[sidx0], rows0, sem0).wait()

            @pl.when(i0 + 2 < NCH)
            def _():
                pltpu.async_copy(src_hbm.at[wid, i0 + 2], sidx0, isem0)

            pltpu.sync_copy(rows0, acc.at[didx.at[i0]], add=True)

            @pl.when(i0 + 2 < NCH)
            def _():
                pltpu.make_async_copy(src_hbm.at[wid, 0], sidx0, isem0).wait()
                _gather(g_hbm, sidx0, rows0, sem0)

            pltpu.make_async_copy(g_hbm.at[sidx1], rows1, sem1).wait()

            @pl.when(i0 + 3 < NCH)
            def _():
                pltpu.async_copy(src_hbm.at[wid, i0 + 3], sidx1, isem1)

            pltpu.sync_copy(rows1, acc.at[didx.at[i0 + 1]], add=True)

            @pl.when(i0 + 3 < NCH)
            def _():
                pltpu.make_async_copy(src_hbm.at[wid, 0], sidx1, isem1).wait()
                _gather(g_hbm, sidx1, rows1, sem1)

            return carry

        lax.fori_loop(0, (NCH - 1) // 2, body, 0)
        pltpu.make_async_copy(g_hbm.at[sidx0], rows0, sem0).wait()
        pltpu.sync_copy(rows0, acc.at[didx.at[NCH - 1]], add=True)

        plsc.subcore_barrier()
        pltpu.sync_copy(acc.at[pl.ds(sid * RS, RS)],
                        out_hbm.at[cid, pl.ds(sid * RS, RS)])

    return sc_scatter


_sc_scatter_rows = _make_sc_scatter(D)


_HROW = NPAD // D


def _make_sc_deg():
    mesh = plsc.VectorSubcoreMesh(core_axis_name="c", subcore_axis_name="s")

    @functools.partial(
        pl.kernel,
        out_type=jax.ShapeDtypeStruct((NC * NS, _HROW, D), jnp.float32),
        mesh=mesh,
        compiler_params=pltpu.CompilerParams(needs_layout_passes=False),
        scratch_types=[
            pltpu.VMEM((EW,), jnp.int32),
            pltpu.VMEM((_HROW, D), jnp.float32),
        ],
    )
    def sc_deg(dst_hbm, zero_hbm, out_hbm, didx, hist):
        cid = lax.axis_index("c")
        sid = lax.axis_index("s")
        wid = cid * NS + sid
        pltpu.sync_copy(zero_hbm.at[pl.ds(0, _HROW)], hist)
        pltpu.sync_copy(dst_hbm.at[pl.ds(wid * EW, EW)], didx)

        def body(k, carry):
            d = didx[pl.ds(16 * k, 16)]
            cnt, last = plsc.scan_count(d)
            row = lax.shift_right_logical(d, 7)
            col = lax.bitwise_and(d, 127)
            plsc.addupdate_scatter(hist, [row, col],
                                   cnt.astype(jnp.float32), mask=last)
            return carry

        lax.fori_loop(0, EW // 16, body, 0)
        pltpu.sync_copy(hist, out_hbm.at[wid])

    return sc_deg


_sc_deg = _make_sc_deg()


def _tc_dinv(degp):
    def body(degp_ref, out_ref):
        acc = degp_ref[0]
        for t in range(1, NC * NS):
            acc = acc + degp_ref[t]
        out_ref[...] = lax.rsqrt(acc + 1.0)

    return pl.pallas_call(
        body,
        out_shape=jax.ShapeDtypeStruct((_HROW, D), jnp.float32),
    )(degp)


def _tc_first(x, fcW, fcb, W0, dinv):
    def body(x_ref, fcW_ref, fcb_ref, W0_ref, dinv_ref, g_ref):
        h = jnp.dot(x_ref[...], fcW_ref[...],
                    preferred_element_type=jnp.float32) + fcb_ref[...]
        p = jnp.dot(h, W0_ref[...], preferred_element_type=jnp.float32)
        g_ref[...] = dinv_ref[...] * p

    return pl.pallas_call(
        body,
        grid=(NBLK,),
        in_specs=[
            pl.BlockSpec((MBLK, D), lambda i: (i, 0)),
            pl.BlockSpec((D, D), lambda i: (0, 0)),
            pl.BlockSpec((1, D), lambda i: (0, 0)),
            pl.BlockSpec((D, D), lambda i: (0, 0)),
            pl.BlockSpec((MBLK, 1), lambda i: (i, 0)),
        ],
        out_specs=pl.BlockSpec((MBLK, D), lambda i: (i, 0)),
        out_shape=jax.ShapeDtypeStruct((N, D), jnp.float32),
    )(x, fcW, fcb, W0, dinv)


def _tc_layer(Sp, g, dinv, b, W):
    def body(sp_ref, g_ref, dinv_ref, b_ref, W_ref, out_ref):
        h = dinv_ref[...] * (sp_ref[0] + sp_ref[1] + g_ref[...]) + b_ref[...]
        out_ref[...] = dinv_ref[...] * jnp.dot(
            h, W_ref[...], preferred_element_type=jnp.float32)

    return pl.pallas_call(
        body,
        grid=(NBLK,),
        in_specs=[
            pl.BlockSpec((NC, MBLK, D), lambda i: (0, i, 0)),
            pl.BlockSpec((MBLK, D), lambda i: (i, 0)),
            pl.BlockSpec((MBLK, 1), lambda i: (i, 0)),
            pl.BlockSpec((1, D), lambda i: (0, 0)),
            pl.BlockSpec((D, D), lambda i: (0, 0)),
        ],
        out_specs=pl.BlockSpec((MBLK, D), lambda i: (i, 0)),
        out_shape=jax.ShapeDtypeStruct((N, D), jnp.float32),
    )(Sp, g, dinv, b, W)


def _tc_final(Sp, g, dinv, b, batch3):
    def body(sp_ref, g_ref, dinv_ref, b_ref, batch_ref, out_ref):
        i = pl.program_id(0)
        h = dinv_ref[...] * (sp_ref[0] + sp_ref[1] + g_ref[...]) + b_ref[...]
        brow = batch_ref[0]
        oh = (brow == lax.broadcasted_iota(jnp.int32, (G, MBLK), 0)
              ).astype(jnp.float32)
        contrib = jnp.dot(oh, h, preferred_element_type=jnp.float32)

        @pl.when(i == 0)
        def _():
            out_ref[...] = jnp.zeros_like(out_ref)

        out_ref[...] += contrib

    return pl.pallas_call(
        body,
        grid=(NBLK,),
        in_specs=[
            pl.BlockSpec((NC, MBLK, D), lambda i: (0, i, 0)),
            pl.BlockSpec((MBLK, D), lambda i: (i, 0)),
            pl.BlockSpec((MBLK, 1), lambda i: (i, 0)),
            pl.BlockSpec((1, D), lambda i: (0, 0)),
            pl.BlockSpec((1, 1, MBLK), lambda i: (i, 0, 0)),
        ],
        out_specs=pl.BlockSpec((G, D), lambda i: (0, 0)),
        out_shape=jax.ShapeDtypeStruct((G, D), jnp.float32),
    )(Sp, g, dinv, b, batch3)


def kernel(x, edge_index, batch, fc_W, fc_b, W0, b0, W1, b1, W2, b2):
    src = edge_index[0].reshape(NC * NS, NCH, CHUNK)
    dst = edge_index[1].reshape(NC * NS, NCH, CHUNK)
    zeroD = jnp.zeros((RS, D), jnp.float32)

    degp = _sc_deg(edge_index[1], zeroD)
    dinv = _tc_dinv(degp).reshape(NPAD, 1)[:N]
    g0 = _tc_first(x, fc_W, fc_b.reshape(1, D), W0, dinv)
    Sp = _sc_scatter_rows(g0, src, dst, zeroD)
    g1 = _tc_layer(Sp, g0, dinv, b0.reshape(1, D), W1)
    Sp = _sc_scatter_rows(g1, src, dst, zeroD)
    g2 = _tc_layer(Sp, g1, dinv, b1.reshape(1, D), W2)
    Sp = _sc_scatter_rows(g2, src, dst, zeroD)
    return _tc_final(Sp, g2, dinv, b2.reshape(1, D),
                     batch.reshape(NBLK, 1, MBLK))
